# baseline (speedup 1.0000x reference)
import jax
import jax.numpy as jnp
from jax import lax
from jax.experimental import pallas as pl
from jax.experimental.pallas import tpu as pltpu

_BT = 1024


def _out_copies(o_hbm, obuf, sem, j):
    b = lax.rem(j, 2)
    half = _BT // 2
    c0 = pltpu.make_async_copy(
        obuf.at[b, pl.ds(0, half), :],
        o_hbm.at[pl.ds(j * _BT, half), :],
        sem.at[b, 0])
    c1 = pltpu.make_async_copy(
        obuf.at[b, pl.ds(half, half), :],
        o_hbm.at[pl.ds(j * _BT + half, half), :],
        sem.at[b, 1])
    return c0, c1


def _lora_body(slot_ref, x1_ref, x2_ref, a_ref, b_ref, o_hbm, obuf, sem):
    i = pl.program_id(0)
    n = pl.num_programs(0)
    r = a_ref.shape[1] // 8
    dh = x1_ref.shape[1]

    @pl.when(i >= 2)
    def _():
        c0, c1 = _out_copies(o_hbm, obuf, sem, i - 2)
        c0.wait()
        c1.wait()

    h = jnp.dot(x1_ref[...].astype(jnp.bfloat16), a_ref[:dh, :],
                preferred_element_type=jnp.float32)
    h += jnp.dot(x2_ref[...].astype(jnp.bfloat16), a_ref[dh:, :],
                 preferred_element_type=jnp.float32)
    col_slot = jax.lax.broadcasted_iota(jnp.int32, h.shape, 1) // r
    mask = slot_ref[...] == col_slot
    hb = jnp.where(mask, h, 0.0).astype(jnp.bfloat16)
    y = jnp.dot(hb, b_ref[...], preferred_element_type=jnp.float32)

    @pl.when(lax.rem(i, 2) == 0)
    def _():
        obuf[0, :, :] = y

    @pl.when(lax.rem(i, 2) == 1)
    def _():
        obuf[1, :, :] = y

    c0, c1 = _out_copies(o_hbm, obuf, sem, i)
    c0.start()
    c1.start()

    @pl.when(i == n - 1)
    def _():
        p0, p1 = _out_copies(o_hbm, obuf, sem, i - 1)
        p0.wait()
        p1.wait()
        q0, q1 = _out_copies(o_hbm, obuf, sem, i)
        q0.wait()
        q1.wait()


def kernel(x, token_to_slot, lora_a, lora_b, lora_scaling):
    T, D = x.shape
    E, _, R = lora_a.shape
    Dout = lora_b.shape[-1]
    Dh = D // 2
    a_cat = jnp.transpose(lora_a, (1, 0, 2)).reshape(D, E * R).astype(jnp.bfloat16)
    b_cat = (lora_b * lora_scaling[:, None, None]).reshape(E * R, Dout)
    b_cat = b_cat.astype(jnp.bfloat16)
    slot2 = token_to_slot.reshape(T, 1)
    return pl.pallas_call(
        _lora_body,
        grid=(T // _BT,),
        in_specs=[
            pl.BlockSpec((_BT, 1), lambda i: (i, 0)),
            pl.BlockSpec((_BT, Dh), lambda i: (i, 0)),
            pl.BlockSpec((_BT, Dh), lambda i: (i, 1)),
            pl.BlockSpec((D, E * R), lambda i: (0, 0)),
            pl.BlockSpec((E * R, Dout), lambda i: (0, 0)),
        ],
        out_specs=pl.BlockSpec(memory_space=pl.ANY),
        out_shape=jax.ShapeDtypeStruct((T, Dout), x.dtype),
        scratch_shapes=[
            pltpu.VMEM((2, _BT, Dout), jnp.float32),
            pltpu.SemaphoreType.DMA((2, 2)),
        ],
    )(slot2, x, x, a_cat, b_cat)
